# baseline (device time: 121029 ns/iter reference)
import jax
import jax.numpy as jnp
from jax import lax
from jax.experimental import pallas as pl
from jax.experimental.pallas import tpu as pltpu

B = 8
SQ = 8
H = 16
HB = 8
D = 128
SKV_LOCAL = 1024
SKV_HALF = SKV_LOCAL // 2
SCALE = D ** -0.5


def kernel(Q, K, V):
    def body(q_ref, k_ref, v_ref, out_ref,
             o_acc, m_acc, l_acc,
             o_rcv1, m_rcv1, l_rcv1,
             o_rcv2, m_rcv2, l_rcv2,
             khs, vhs, gather_sems,
             send_sems, recv_sems):
        b = pl.program_id(0)
        hb = pl.program_id(1)

        h0 = hb * HB
        gathers = []
        for h in range(HB):
            ck = pltpu.make_async_copy(
                k_ref.at[0, :, h, :], khs.at[h], gather_sems.at[2 * h])
            cv = pltpu.make_async_copy(
                v_ref.at[0, :, h, :], vhs.at[h], gather_sems.at[2 * h + 1])
            ck.start()
            cv.start()
            gathers.append((ck, cv))
        for h in range(HB):
            ck, cv = gathers[h]
            ck.wait()
            cv.wait()
            qh = q_ref[0, :, h, :].astype(jnp.bfloat16)
            kh = khs[h].astype(jnp.bfloat16)
            vh = vhs[h].astype(jnp.bfloat16)
            s = lax.dot_general(
                qh, kh, (((1,), (1,)), ((), ())),
                preferred_element_type=jnp.float32,
            ) * SCALE
            m = jnp.max(s, axis=1)
            p = jnp.exp(s - m[:, None])
            l = jnp.sum(p, axis=1)
            o = lax.dot_general(
                p.astype(jnp.bfloat16), vh, (((1,), (0,)), ((), ())),
                preferred_element_type=jnp.float32,
            )
            o_acc[b, h0 + h, :, :] = o
            m_acc[b, pl.ds(h0 + h, 1), :] = m.reshape(1, SQ)
            l_acc[b, pl.ds(h0 + h, 1), :] = l.reshape(1, SQ)

        @pl.when(jnp.logical_and(b == B - 1, hb == H // HB - 1))
        def _():
            my_x = lax.axis_index("x")
            my_y = lax.axis_index("y")

            def exchange(partner, dsts, sem0):
                copies = [
                    pltpu.make_async_remote_copy(
                        src_ref=src, dst_ref=dst,
                        send_sem=send_sems.at[sem0 + j],
                        recv_sem=recv_sems.at[sem0 + j],
                        device_id=partner,
                        device_id_type=pl.DeviceIdType.MESH,
                    )
                    for j, (src, dst) in enumerate(
                        zip([o_acc, m_acc, l_acc], dsts)
                    )
                ]
                for c in copies:
                    c.start()
                for c in copies:
                    c.wait()

            exchange((1 - my_x, my_y), [o_rcv1, m_rcv1, l_rcv1], 0)
            m1 = jnp.maximum(m_acc[...], m_rcv1[...])
            wa = jnp.exp(m_acc[...] - m1)
            wb = jnp.exp(m_rcv1[...] - m1)
            l1 = wa * l_acc[...] + wb * l_rcv1[...]
            o1 = wa[..., None] * o_acc[...] + wb[..., None] * o_rcv1[...]
            m_acc[...] = m1
            l_acc[...] = l1
            o_acc[...] = o1

            exchange((my_x, 1 - my_y), [o_rcv2, m_rcv2, l_rcv2], 3)
            m2 = jnp.maximum(m_acc[...], m_rcv2[...])
            wc = jnp.exp(m_acc[...] - m2)
            wd = jnp.exp(m_rcv2[...] - m2)
            l2 = wc * l_acc[...] + wd * l_rcv2[...]
            o2 = (wc[..., None] * o_acc[...]
                  + wd[..., None] * o_rcv2[...]) / l2[..., None]
            out = o2.transpose(0, 2, 1, 3)
            out_ref[...] = out.astype(jnp.float32)

    def kv_map(bi, hbi):
        return (bi, lax.axis_index("x"), hbi, 0)

    return pl.pallas_call(
        body,
        grid=(B, H // HB),
        in_specs=[
            pl.BlockSpec((1, SQ, HB, D), lambda bi, hbi: (bi, 0, hbi, 0)),
            pl.BlockSpec((1, SKV_HALF, HB, D), kv_map),
            pl.BlockSpec((1, SKV_HALF, HB, D), kv_map),
        ],
        out_specs=pl.BlockSpec((B, SQ, H, D), lambda bi, hbi: (0, 0, 0, 0)),
        out_shape=jax.ShapeDtypeStruct((B, SQ, H, D), jnp.float32),
        scratch_shapes=[
            pltpu.VMEM((B, H, SQ, D), jnp.float32),
            pltpu.VMEM((B, H, SQ), jnp.float32),
            pltpu.VMEM((B, H, SQ), jnp.float32),
            pltpu.VMEM((B, H, SQ, D), jnp.float32),
            pltpu.VMEM((B, H, SQ), jnp.float32),
            pltpu.VMEM((B, H, SQ), jnp.float32),
            pltpu.VMEM((B, H, SQ, D), jnp.float32),
            pltpu.VMEM((B, H, SQ), jnp.float32),
            pltpu.VMEM((B, H, SQ), jnp.float32),
            pltpu.VMEM((HB, SKV_HALF, D), jnp.float32),
            pltpu.VMEM((HB, SKV_HALF, D), jnp.float32),
            pltpu.SemaphoreType.DMA((2 * HB,)),
            pltpu.SemaphoreType.DMA((6,)),
            pltpu.SemaphoreType.DMA((6,)),
        ],
    )(Q, K, V)


# device time: 119866 ns/iter; 1.0097x vs baseline; 1.0097x over previous
import jax
import jax.numpy as jnp
from jax import lax
from jax.experimental import pallas as pl
from jax.experimental.pallas import tpu as pltpu

B = 8
SQ = 8
H = 16
HB = 8
D = 128
SKV_LOCAL = 1024
SKV_HALF = SKV_LOCAL // 2
SCALE = D ** -0.5


def kernel(Q, K, V):
    def body(q_ref, k_ref, v_ref, out_ref,
             o_acc, m_acc, l_acc,
             o_rcv1, m_rcv1, l_rcv1,
             o_rcv2, m_rcv2, l_rcv2,
             send_sems, recv_sems):
        b = pl.program_id(0)
        hb = pl.program_id(1)

        h0 = hb * HB
        for h in range(HB):
            qh = q_ref[0, :, h, :].astype(jnp.bfloat16)
            kh = k_ref[0, :, h, :].astype(jnp.bfloat16)
            vh = v_ref[0, :, h, :].astype(jnp.bfloat16)
            st = lax.dot_general(
                kh, qh, (((1,), (1,)), ((), ())),
                preferred_element_type=jnp.float32,
            ) * SCALE
            m = jnp.max(st, axis=0)
            pt = jnp.exp(st - m[None, :])
            l = jnp.sum(pt, axis=0)
            o = lax.dot_general(
                pt.astype(jnp.bfloat16), vh, (((0,), (0,)), ((), ())),
                preferred_element_type=jnp.float32,
            )
            o_acc[b, h0 + h, :, :] = o
            m_acc[b, pl.ds(h0 + h, 1), :] = m.reshape(1, SQ)
            l_acc[b, pl.ds(h0 + h, 1), :] = l.reshape(1, SQ)

        @pl.when(jnp.logical_and(b == B - 1, hb == H // HB - 1))
        def _():
            my_x = lax.axis_index("x")
            my_y = lax.axis_index("y")

            def exchange(partner, dsts, sem0):
                copies = [
                    pltpu.make_async_remote_copy(
                        src_ref=src, dst_ref=dst,
                        send_sem=send_sems.at[sem0 + j],
                        recv_sem=recv_sems.at[sem0 + j],
                        device_id=partner,
                        device_id_type=pl.DeviceIdType.MESH,
                    )
                    for j, (src, dst) in enumerate(
                        zip([o_acc, m_acc, l_acc], dsts)
                    )
                ]
                for c in copies:
                    c.start()
                for c in copies:
                    c.wait()

            exchange((1 - my_x, my_y), [o_rcv1, m_rcv1, l_rcv1], 0)
            m1 = jnp.maximum(m_acc[...], m_rcv1[...])
            wa = jnp.exp(m_acc[...] - m1)
            wb = jnp.exp(m_rcv1[...] - m1)
            l1 = wa * l_acc[...] + wb * l_rcv1[...]
            o1 = wa[..., None] * o_acc[...] + wb[..., None] * o_rcv1[...]
            m_acc[...] = m1
            l_acc[...] = l1
            o_acc[...] = o1

            exchange((my_x, 1 - my_y), [o_rcv2, m_rcv2, l_rcv2], 3)
            m2 = jnp.maximum(m_acc[...], m_rcv2[...])
            wc = jnp.exp(m_acc[...] - m2)
            wd = jnp.exp(m_rcv2[...] - m2)
            l2 = wc * l_acc[...] + wd * l_rcv2[...]
            o2 = (wc[..., None] * o_acc[...]
                  + wd[..., None] * o_rcv2[...]) / l2[..., None]
            out = o2.transpose(0, 2, 1, 3)
            out_ref[...] = out.astype(jnp.float32)

    def kv_map(bi, hbi):
        return (bi, lax.axis_index("x"), hbi, 0)

    return pl.pallas_call(
        body,
        grid=(B, H // HB),
        in_specs=[
            pl.BlockSpec((1, SQ, HB, D), lambda bi, hbi: (bi, 0, hbi, 0)),
            pl.BlockSpec((1, SKV_HALF, HB, D), kv_map),
            pl.BlockSpec((1, SKV_HALF, HB, D), kv_map),
        ],
        out_specs=pl.BlockSpec((B, SQ, H, D), lambda bi, hbi: (0, 0, 0, 0)),
        out_shape=jax.ShapeDtypeStruct((B, SQ, H, D), jnp.float32),
        scratch_shapes=[
            pltpu.VMEM((B, H, SQ, D), jnp.float32),
            pltpu.VMEM((B, H, SQ), jnp.float32),
            pltpu.VMEM((B, H, SQ), jnp.float32),
            pltpu.VMEM((B, H, SQ, D), jnp.float32),
            pltpu.VMEM((B, H, SQ), jnp.float32),
            pltpu.VMEM((B, H, SQ), jnp.float32),
            pltpu.VMEM((B, H, SQ, D), jnp.float32),
            pltpu.VMEM((B, H, SQ), jnp.float32),
            pltpu.VMEM((B, H, SQ), jnp.float32),
            pltpu.SemaphoreType.DMA((6,)),
            pltpu.SemaphoreType.DMA((6,)),
        ],
    )(Q, K, V)


# device time: 84668 ns/iter; 1.4295x vs baseline; 1.4157x over previous
import jax
import jax.numpy as jnp
from jax import lax
from jax.experimental import pallas as pl
from jax.experimental.pallas import tpu as pltpu

B = 8
SQ = 8
H = 16
D = 128
SKV_LOCAL = 1024
SKV_HALF = SKV_LOCAL // 2
CHUNK = 256
NCHUNK = B * SKV_HALF // CHUNK
SCALE = D ** -0.5


def kernel(Q, K, V):
    def body(q_ref, k_hbm, v_hbm, out_ref,
             ktr, vbuf,
             o_acc, m_acc, l_acc,
             o_cmb, m_cmb, l_cmb, o_snd,
             o_rcv1, m_rcv1, l_rcv1,
             o_rcv2, m_rcv2, l_rcv2,
             load_sems, send_sems, recv_sems):
        my_x = lax.axis_index("x")
        my_y = lax.axis_index("y")
        k0 = my_x * SKV_HALF

        def load(c, slot):
            b, j = c // 2, c % 2
            ks = k0 + j * CHUNK
            copies = []
            for h in range(H):
                ck = pltpu.make_async_copy(
                    k_hbm.at[b, pl.ds(ks, CHUNK), h, :], ktr.at[slot, h],
                    load_sems.at[slot, 0, h])
                ck.start()
                copies.append(ck)
            cv = pltpu.make_async_copy(
                v_hbm.at[b, pl.ds(ks, CHUNK), :, :], vbuf.at[slot],
                load_sems.at[slot, 1, 0])
            cv.start()
            copies.append(cv)
            return copies

        loads = {0: load(0, 0)}
        for c in range(NCHUNK):
            slot = c % 2
            b, j = c // 2, c % 2
            for cp in loads.pop(c):
                cp.wait()
            if c + 1 < NCHUNK:
                loads[c + 1] = load(c + 1, (c + 1) % 2)
            for h in range(H):
                qh = q_ref[b, :, h, :].astype(jnp.bfloat16)
                kh = ktr[slot, h].astype(jnp.bfloat16)
                vh = vbuf[slot, :, h, :].astype(jnp.bfloat16)
                s = lax.dot_general(
                    qh, kh, (((1,), (1,)), ((), ())),
                    preferred_element_type=jnp.float32,
                ) * SCALE
                m = jnp.max(s, axis=1)
                p = jnp.exp(s - m[:, None])
                l = jnp.sum(p, axis=1)
                o = lax.dot_general(
                    p.astype(jnp.bfloat16), vh, (((1,), (0,)), ((), ())),
                    preferred_element_type=jnp.float32,
                )
                o_acc[j, b, h, :, :] = o
                m_acc[j, b, pl.ds(h, 1), :] = m.reshape(1, SQ)
                l_acc[j, b, pl.ds(h, 1), :] = l.reshape(1, SQ)

        HB2 = B // 2

        def exchange(partner_a, partner_b, dsts, sem0):
            copies = []
            for g, (lo, partner) in enumerate(
                [(0, partner_a), (HB2, partner_b)]
            ):
                for i, (src, dst) in enumerate(
                    zip([o_snd, m_cmb, l_cmb], dsts)
                ):
                    copies.append(pltpu.make_async_remote_copy(
                        src_ref=src.at[pl.ds(lo, HB2)],
                        dst_ref=dst.at[pl.ds(lo, HB2)],
                        send_sem=send_sems.at[sem0 + 3 * g + i],
                        recv_sem=recv_sems.at[sem0 + 3 * g + i],
                        device_id=partner,
                        device_id_type=pl.DeviceIdType.MESH,
                    ))
            for cp in copies:
                cp.start()
            for cp in copies:
                cp.wait()

        m0 = jnp.maximum(m_acc[0], m_acc[1])
        w0 = jnp.exp(m_acc[0] - m0)
        w1 = jnp.exp(m_acc[1] - m0)
        m_cmb[...] = m0
        l_cmb[...] = w0 * l_acc[0] + w1 * l_acc[1]
        o0 = w0[..., None] * o_acc[0] + w1[..., None] * o_acc[1]
        o_cmb[...] = o0
        o_snd[...] = o0.astype(jnp.bfloat16)

        exchange((1 - my_x, my_y), (my_x, 1 - my_y),
                 [o_rcv1, m_rcv1, l_rcv1], 0)
        m1 = jnp.maximum(m_cmb[...], m_rcv1[...])
        wa = jnp.exp(m_cmb[...] - m1)
        wb = jnp.exp(m_rcv1[...] - m1)
        l1 = wa * l_cmb[...] + wb * l_rcv1[...]
        o1 = (wa[..., None] * o_cmb[...]
              + wb[..., None] * o_rcv1[...].astype(jnp.float32))
        m_cmb[...] = m1
        l_cmb[...] = l1
        o_cmb[...] = o1
        o_snd[...] = o1.astype(jnp.bfloat16)

        exchange((my_x, 1 - my_y), (1 - my_x, my_y),
                 [o_rcv2, m_rcv2, l_rcv2], 6)
        m2 = jnp.maximum(m_cmb[...], m_rcv2[...])
        wc = jnp.exp(m_cmb[...] - m2)
        wd = jnp.exp(m_rcv2[...] - m2)
        l2 = wc * l_cmb[...] + wd * l_rcv2[...]
        o2 = (wc[..., None] * o_cmb[...]
              + wd[..., None] * o_rcv2[...].astype(jnp.float32)
              ) / l2[..., None]
        out = o2.transpose(0, 2, 1, 3)
        out_ref[...] = out.astype(jnp.float32)

    return pl.pallas_call(
        body,
        in_specs=[
            pl.BlockSpec(memory_space=pltpu.VMEM),
            pl.BlockSpec(memory_space=pl.ANY),
            pl.BlockSpec(memory_space=pl.ANY),
        ],
        out_specs=pl.BlockSpec(memory_space=pltpu.VMEM),
        out_shape=jax.ShapeDtypeStruct((B, SQ, H, D), jnp.float32),
        scratch_shapes=[
            pltpu.VMEM((2, H, CHUNK, D), jnp.float32),
            pltpu.VMEM((2, CHUNK, H, D), jnp.float32),
            pltpu.VMEM((2, B, H, SQ, D), jnp.float32),
            pltpu.VMEM((2, B, H, SQ), jnp.float32),
            pltpu.VMEM((2, B, H, SQ), jnp.float32),
            pltpu.VMEM((B, H, SQ, D), jnp.float32),
            pltpu.VMEM((B, H, SQ), jnp.float32),
            pltpu.VMEM((B, H, SQ), jnp.float32),
            pltpu.VMEM((B, H, SQ, D), jnp.bfloat16),
            pltpu.VMEM((B, H, SQ, D), jnp.bfloat16),
            pltpu.VMEM((B, H, SQ), jnp.float32),
            pltpu.VMEM((B, H, SQ), jnp.float32),
            pltpu.VMEM((B, H, SQ, D), jnp.bfloat16),
            pltpu.VMEM((B, H, SQ), jnp.float32),
            pltpu.VMEM((B, H, SQ), jnp.float32),
            pltpu.SemaphoreType.DMA((2, 2, H)),
            pltpu.SemaphoreType.DMA((12,)),
            pltpu.SemaphoreType.DMA((12,)),
        ],
    )(Q, K, V)
